# baseline (speedup 1.0000x reference)
import functools

import jax
import jax.numpy as jnp
from jax import lax
from jax.experimental import pallas as pl
from jax.experimental.pallas import tpu as pltpu
from jax.experimental.pallas import tpu_sc as plsc

N = 10000
E = 320000
F_IN = 128
H = 64
C = 10
B = 64

NC = 2
NS = 16
L = 16
NW = NC * NS
K = 128
NCH = 80
J = 5
NSUP = NCH // J
EPAD = NW * NCH * K
NPAD = N + 8

ROWS_BLK = 1000
GRID = N // ROWS_BLK


def _deg_body(dst_hbm, z_hbm, out_hbm, acc, ones, dbufs, semd):
    cid = lax.axis_index("c")
    sid = lax.axis_index("s")
    w = cid * NS + sid
    one16 = jnp.ones((L,), jnp.float32)

    @pl.when(sid == 0)
    def _():
        pltpu.sync_copy(z_hbm, acc)

    @pl.loop(0, K)
    def _(i):
        ones[i] = one16

    pltpu.sync_copy(dst_hbm.at[w], dbufs)
    plsc.subcore_barrier()

    def drain_one():
        pltpu.make_async_copy(z_hbm.at[pl.ds(0, K)], ones, semd).wait()

    @pl.loop(0, NCH)
    def _(c):
        @pl.when(c >= 8)
        def _():
            drain_one()
        pltpu.async_copy(ones, acc.at[dbufs.at[c]], semd, add=True)

    for _ in range(8):
        drain_one()

    plsc.subcore_barrier()

    @pl.when(sid == 0)
    def _():
        pltpu.sync_copy(acc, out_hbm.at[cid])


@functools.cache
def _deg_call():
    mesh = plsc.VectorSubcoreMesh(
        core_axis_name="c", subcore_axis_name="s",
        num_cores=NC, num_subcores=NS,
    )
    return pl.kernel(
        _deg_body,
        out_type=jax.ShapeDtypeStruct((NC, NPAD, L), jnp.float32),
        mesh=mesh,
        compiler_params=pltpu.CompilerParams(use_tc_tiling_on_sc=False),
        scratch_types=[
            pltpu.VMEM_SHARED((NPAD, L), jnp.float32),
            pltpu.VMEM((K, L), jnp.float32),
            pltpu.VMEM((NCH, K), jnp.int32),
            pltpu.SemaphoreType.DMA,
        ],
    )


def _agg_body(src_hbm, dst_hbm, tp_hbm, z_hbm, out_hbm,
              acc, sbufs, dbufs, rows, semg, sems):
    cid = lax.axis_index("c")
    sid = lax.axis_index("s")
    w = cid * NS + sid

    @pl.when(sid == 0)
    def _():
        pltpu.sync_copy(z_hbm, acc)

    pltpu.sync_copy(src_hbm.at[w], sbufs)
    pltpu.sync_copy(dst_hbm.at[w], dbufs)
    plsc.subcore_barrier()

    def issue_gather(c, slot):
        pltpu.async_copy(tp_hbm.at[sbufs.at[c]], rows.at[slot], semg)

    def wait_gather():
        pltpu.make_async_copy(tp_hbm.at[pl.ds(0, K)], rows.at[0], semg).wait()

    def drain_scatter():
        pltpu.make_async_copy(tp_hbm.at[pl.ds(0, K)], rows.at[0], sems).wait()

    issue_gather(0, 0)
    issue_gather(1, 1)

    @pl.loop(0, NSUP)
    def _(sc):
        for j in range(J):
            c = sc * J + j
            nxt = c + 2

            @pl.when(nxt < NCH)
            def _():
                @pl.when(nxt >= J)
                def _():
                    drain_scatter()
                issue_gather(nxt, (j + 2) % J)

            wait_gather()
            pltpu.async_copy(rows.at[j], acc.at[dbufs.at[c]], sems, add=True)

    for _ in range(J):
        drain_scatter()

    plsc.subcore_barrier()

    @pl.when(sid == 0)
    def _():
        pltpu.sync_copy(acc, out_hbm.at[cid])


@functools.cache
def _agg_call():
    mesh = plsc.VectorSubcoreMesh(
        core_axis_name="c", subcore_axis_name="s",
        num_cores=NC, num_subcores=NS,
    )
    return pl.kernel(
        _agg_body,
        out_type=jax.ShapeDtypeStruct((NC, NPAD, H), jnp.float32),
        mesh=mesh,
        compiler_params=pltpu.CompilerParams(use_tc_tiling_on_sc=False),
        scratch_types=[
            pltpu.VMEM_SHARED((NPAD, H), jnp.float32),
            pltpu.VMEM((NCH, K), jnp.int32),
            pltpu.VMEM((NCH, K), jnp.int32),
            pltpu.VMEM((J, K, H), jnp.float32),
            pltpu.SemaphoreType.DMA,
            pltpu.SemaphoreType.DMA,
        ],
    )


def _mm_body(x_ref, w_ref, o_ref):
    o_ref[...] = jnp.dot(x_ref[...], w_ref[...],
                         preferred_element_type=jnp.float32)


_mm_call = pl.pallas_call(
    _mm_body,
    grid=(GRID,),
    in_specs=[
        pl.BlockSpec((ROWS_BLK, F_IN), lambda i: (i, 0)),
        pl.BlockSpec((F_IN, H), lambda i: (0, 0)),
    ],
    out_specs=pl.BlockSpec((ROWS_BLK, H), lambda i: (i, 0)),
    out_shape=jax.ShapeDtypeStruct((N, H), jnp.float32),
)


def _scale_body(p_ref, t_ref, dis_ref, tp_ref):
    indeg = p_ref[0, :, 0:1] + p_ref[1, :, 0:1]
    dis = lax.rsqrt(indeg + 1.0)
    dis_ref[...] = dis
    tp_ref[...] = t_ref[...] * dis


_scale_call = pl.pallas_call(
    _scale_body,
    grid=(GRID,),
    in_specs=[
        pl.BlockSpec((NC, ROWS_BLK, L), lambda i: (0, i, 0)),
        pl.BlockSpec((ROWS_BLK, H), lambda i: (i, 0)),
    ],
    out_specs=[
        pl.BlockSpec((ROWS_BLK, 1), lambda i: (i, 0)),
        pl.BlockSpec((ROWS_BLK, H), lambda i: (i, 0)),
    ],
    out_shape=[
        jax.ShapeDtypeStruct((N, 1), jnp.float32),
        jax.ShapeDtypeStruct((NPAD, H), jnp.float32),
    ],
)


def _bnd_body(p_ref, tp_ref, dis_ref, b_ref, w_ref, o_ref):
    dis = dis_ref[...]
    h = jnp.tanh(dis * (p_ref[0] + p_ref[1] + tp_ref[...]) + b_ref[...])
    o_ref[...] = jnp.dot(h, w_ref[...],
                         preferred_element_type=jnp.float32) * dis


_bnd_call = pl.pallas_call(
    _bnd_body,
    grid=(GRID,),
    in_specs=[
        pl.BlockSpec((NC, ROWS_BLK, H), lambda i: (0, i, 0)),
        pl.BlockSpec((ROWS_BLK, H), lambda i: (i, 0)),
        pl.BlockSpec((ROWS_BLK, 1), lambda i: (i, 0)),
        pl.BlockSpec((1, H), lambda i: (0, 0)),
        pl.BlockSpec((H, H), lambda i: (0, 0)),
    ],
    out_specs=pl.BlockSpec((ROWS_BLK, H), lambda i: (i, 0)),
    out_shape=jax.ShapeDtypeStruct((NPAD, H), jnp.float32),
)


def _fin_body(p_ref, tp_ref, dis_ref, b_ref, batch_ref, wf_ref, bf_ref,
              o_ref, acc):
    i = pl.program_id(0)

    @pl.when(i == 0)
    def _():
        acc[...] = jnp.zeros_like(acc)

    h = jnp.tanh(dis_ref[...] * (p_ref[0] + p_ref[1] + tp_ref[...])
                 + b_ref[...])
    seg = lax.broadcasted_iota(jnp.int32, (B, ROWS_BLK), 0)
    onehot = (seg == batch_ref[0]).astype(jnp.float32)
    acc[...] += jnp.dot(onehot, h, preferred_element_type=jnp.float32)

    @pl.when(i == GRID - 1)
    def _():
        o_ref[...] = jnp.tanh(
            jnp.dot(acc[...], wf_ref[...],
                    preferred_element_type=jnp.float32) + bf_ref[...])


_fin_call = pl.pallas_call(
    _fin_body,
    grid=(GRID,),
    in_specs=[
        pl.BlockSpec((NC, ROWS_BLK, H), lambda i: (0, i, 0)),
        pl.BlockSpec((ROWS_BLK, H), lambda i: (i, 0)),
        pl.BlockSpec((ROWS_BLK, 1), lambda i: (i, 0)),
        pl.BlockSpec((1, H), lambda i: (0, 0)),
        pl.BlockSpec((1, 1, ROWS_BLK), lambda i: (i, 0, 0)),
        pl.BlockSpec((H, C), lambda i: (0, 0)),
        pl.BlockSpec((1, C), lambda i: (0, 0)),
    ],
    out_specs=pl.BlockSpec((B, C), lambda i: (0, 0)),
    out_shape=jax.ShapeDtypeStruct((B, C), jnp.float32),
    scratch_shapes=[pltpu.VMEM((B, H), jnp.float32)],
)


def kernel(x, edge_index, batch, W1, b1, W2, b2, W3, b3, Wf, bf):
    pad = jnp.full((EPAD - E,), N, jnp.int32)
    srcr = jnp.concatenate([edge_index[0], pad]).reshape(NW, NCH, K)
    dstr = jnp.concatenate([edge_index[1], pad]).reshape(NW, NCH, K)
    batch2 = batch.reshape(GRID, 1, ROWS_BLK)

    z16 = jnp.zeros((NPAD, L), jnp.float32)
    z64 = jnp.zeros((NPAD, H), jnp.float32)

    degp = _deg_call()(dstr, z16)
    t1 = _mm_call(x, W1)
    dis, tp = _scale_call(degp, t1)

    for W_next, b_cur in ((W2, b1), (W3, b2)):
        p = _agg_call()(srcr, dstr, tp, z64)
        tp = _bnd_call(p, tp, dis, b_cur.reshape(1, H), W_next)

    p = _agg_call()(srcr, dstr, tp, z64)
    return _fin_call(p, tp, dis, b3.reshape(1, H), batch2, Wf,
                     bf.reshape(1, C))


# speedup vs baseline: 2.3273x; 2.3273x over previous
import functools

import jax
import jax.numpy as jnp
from jax import lax
from jax.experimental import pallas as pl
from jax.experimental.pallas import tpu as pltpu
from jax.experimental.pallas import tpu_sc as plsc

N = 10000
E = 320000
F_IN = 128
H = 64
C = 10
B = 64

NC = 2
NS = 16
L = 16
NW = NC * NS
EPW = E // NW
K = 80
J = 5
NCH = EPW // K
NSUP = NCH // J

ROWS_BLK = 1000
GRID = N // ROWS_BLK


def _deg_body(dst_hbm, z_hbm, out_hbm, acc, ones, dbufs, semd):
    cid = lax.axis_index("c")
    sid = lax.axis_index("s")
    w = cid * NS + sid
    one16 = jnp.ones((L,), jnp.float32)

    @pl.when(sid == 0)
    def _():
        pltpu.sync_copy(z_hbm, acc)

    @pl.loop(0, K)
    def _(i):
        ones[i] = one16

    pltpu.sync_copy(dst_hbm.at[w], dbufs)
    plsc.subcore_barrier()

    def drain_one():
        pltpu.make_async_copy(z_hbm.at[pl.ds(0, K)], ones, semd).wait()

    @pl.loop(0, NCH)
    def _(c):
        @pl.when(c >= 8)
        def _():
            drain_one()
        pltpu.async_copy(ones, acc.at[dbufs.at[c]], semd, add=True)

    for _ in range(8):
        drain_one()

    plsc.subcore_barrier()

    @pl.when(sid == 0)
    def _():
        pltpu.sync_copy(acc, out_hbm.at[cid])


@functools.cache
def _deg_call():
    mesh = plsc.VectorSubcoreMesh(
        core_axis_name="c", subcore_axis_name="s",
        num_cores=NC, num_subcores=NS,
    )
    return pl.kernel(
        _deg_body,
        out_type=jax.ShapeDtypeStruct((NC, N, L), jnp.float32),
        mesh=mesh,
        compiler_params=pltpu.CompilerParams(use_tc_tiling_on_sc=False),
        scratch_types=[
            pltpu.VMEM_SHARED((N, L), jnp.float32),
            pltpu.VMEM((K, L), jnp.float32),
            pltpu.VMEM((NCH, K), jnp.int32),
            pltpu.SemaphoreType.DMA,
        ],
    )


def _agg_body(src_hbm, dst_hbm, tp_hbm, z_hbm, out_hbm,
              acc, sbufs, dbufs, rows, semg, sems):
    cid = lax.axis_index("c")
    sid = lax.axis_index("s")
    w = cid * NS + sid

    @pl.when(sid == 0)
    def _():
        pltpu.sync_copy(z_hbm, acc)

    pltpu.sync_copy(src_hbm.at[w], sbufs)
    pltpu.sync_copy(dst_hbm.at[w], dbufs)
    plsc.subcore_barrier()

    def issue_gather(c, slot):
        pltpu.async_copy(tp_hbm.at[sbufs.at[c]], rows.at[slot], semg)

    def wait_gather():
        pltpu.make_async_copy(tp_hbm.at[pl.ds(0, K)], rows.at[0], semg).wait()

    def drain_scatter():
        pltpu.make_async_copy(tp_hbm.at[pl.ds(0, K)], rows.at[0], sems).wait()

    issue_gather(0, 0)
    issue_gather(1, 1)

    @pl.loop(0, NSUP)
    def _(sc):
        for j in range(J):
            c = sc * J + j
            nxt = c + 2

            @pl.when(nxt < NCH)
            def _():
                @pl.when(nxt >= J)
                def _():
                    drain_scatter()
                issue_gather(nxt, (j + 2) % J)

            wait_gather()
            pltpu.async_copy(rows.at[j], acc.at[dbufs.at[c]], sems, add=True)

    for _ in range(J):
        drain_scatter()

    plsc.subcore_barrier()

    @pl.when(sid == 0)
    def _():
        pltpu.sync_copy(acc, out_hbm.at[cid])


@functools.cache
def _agg_call():
    mesh = plsc.VectorSubcoreMesh(
        core_axis_name="c", subcore_axis_name="s",
        num_cores=NC, num_subcores=NS,
    )
    return pl.kernel(
        _agg_body,
        out_type=jax.ShapeDtypeStruct((NC, N, H), jnp.float32),
        mesh=mesh,
        compiler_params=pltpu.CompilerParams(use_tc_tiling_on_sc=False),
        scratch_types=[
            pltpu.VMEM_SHARED((N, H), jnp.float32),
            pltpu.VMEM((NCH, K), jnp.int32),
            pltpu.VMEM((NCH, K), jnp.int32),
            pltpu.VMEM((J, K, H), jnp.float32),
            pltpu.SemaphoreType.DMA,
            pltpu.SemaphoreType.DMA,
        ],
    )


def _mm_body(x_ref, w_ref, o_ref):
    o_ref[...] = jnp.dot(x_ref[...], w_ref[...],
                         preferred_element_type=jnp.float32)


_mm_call = pl.pallas_call(
    _mm_body,
    grid=(GRID,),
    in_specs=[
        pl.BlockSpec((ROWS_BLK, F_IN), lambda i: (i, 0)),
        pl.BlockSpec((F_IN, H), lambda i: (0, 0)),
    ],
    out_specs=pl.BlockSpec((ROWS_BLK, H), lambda i: (i, 0)),
    out_shape=jax.ShapeDtypeStruct((N, H), jnp.float32),
)


def _scale_body(p_ref, t_ref, dis_ref, tp_ref):
    indeg = p_ref[0, :, 0:1] + p_ref[1, :, 0:1]
    dis = lax.rsqrt(indeg + 1.0)
    dis_ref[...] = dis
    tp_ref[...] = t_ref[...] * dis


_scale_call = pl.pallas_call(
    _scale_body,
    grid=(GRID,),
    in_specs=[
        pl.BlockSpec((NC, ROWS_BLK, L), lambda i: (0, i, 0)),
        pl.BlockSpec((ROWS_BLK, H), lambda i: (i, 0)),
    ],
    out_specs=[
        pl.BlockSpec((ROWS_BLK, 1), lambda i: (i, 0)),
        pl.BlockSpec((ROWS_BLK, H), lambda i: (i, 0)),
    ],
    out_shape=[
        jax.ShapeDtypeStruct((N, 1), jnp.float32),
        jax.ShapeDtypeStruct((N, H), jnp.float32),
    ],
)


def _bnd_body(p_ref, tp_ref, dis_ref, b_ref, w_ref, o_ref):
    dis = dis_ref[...]
    h = jnp.tanh(dis * (p_ref[0] + p_ref[1] + tp_ref[...]) + b_ref[...])
    o_ref[...] = jnp.dot(h, w_ref[...],
                         preferred_element_type=jnp.float32) * dis


_bnd_call = pl.pallas_call(
    _bnd_body,
    grid=(GRID,),
    in_specs=[
        pl.BlockSpec((NC, ROWS_BLK, H), lambda i: (0, i, 0)),
        pl.BlockSpec((ROWS_BLK, H), lambda i: (i, 0)),
        pl.BlockSpec((ROWS_BLK, 1), lambda i: (i, 0)),
        pl.BlockSpec((1, H), lambda i: (0, 0)),
        pl.BlockSpec((H, H), lambda i: (0, 0)),
    ],
    out_specs=pl.BlockSpec((ROWS_BLK, H), lambda i: (i, 0)),
    out_shape=jax.ShapeDtypeStruct((N, H), jnp.float32),
)


def _fin_body(p_ref, tp_ref, dis_ref, b_ref, batch_ref, wf_ref, bf_ref,
              o_ref, acc):
    i = pl.program_id(0)

    @pl.when(i == 0)
    def _():
        acc[...] = jnp.zeros_like(acc)

    h = jnp.tanh(dis_ref[...] * (p_ref[0] + p_ref[1] + tp_ref[...])
                 + b_ref[...])
    seg = lax.broadcasted_iota(jnp.int32, (B, ROWS_BLK), 0)
    onehot = (seg == batch_ref[0]).astype(jnp.float32)
    acc[...] += jnp.dot(onehot, h, preferred_element_type=jnp.float32)

    @pl.when(i == GRID - 1)
    def _():
        o_ref[...] = jnp.tanh(
            jnp.dot(acc[...], wf_ref[...],
                    preferred_element_type=jnp.float32) + bf_ref[...])


_fin_call = pl.pallas_call(
    _fin_body,
    grid=(GRID,),
    in_specs=[
        pl.BlockSpec((NC, ROWS_BLK, H), lambda i: (0, i, 0)),
        pl.BlockSpec((ROWS_BLK, H), lambda i: (i, 0)),
        pl.BlockSpec((ROWS_BLK, 1), lambda i: (i, 0)),
        pl.BlockSpec((1, H), lambda i: (0, 0)),
        pl.BlockSpec((1, 1, ROWS_BLK), lambda i: (i, 0, 0)),
        pl.BlockSpec((H, C), lambda i: (0, 0)),
        pl.BlockSpec((1, C), lambda i: (0, 0)),
    ],
    out_specs=pl.BlockSpec((B, C), lambda i: (0, 0)),
    out_shape=jax.ShapeDtypeStruct((B, C), jnp.float32),
    scratch_shapes=[pltpu.VMEM((B, H), jnp.float32)],
)


def kernel(x, edge_index, batch, W1, b1, W2, b2, W3, b3, Wf, bf):
    srcr = edge_index[0].reshape(NW, NCH, K)
    dstr = edge_index[1].reshape(NW, NCH, K)
    batch2 = batch.reshape(GRID, 1, ROWS_BLK)

    z16 = jnp.zeros((N, L), jnp.float32)
    z64 = jnp.zeros((N, H), jnp.float32)

    degp = _deg_call()(dstr, z16)
    t1 = _mm_call(x, W1)
    dis, tp = _scale_call(degp, t1)

    for W_next, b_cur in ((W2, b1), (W3, b2)):
        p = _agg_call()(srcr, dstr, tp, z64)
        tp = _bnd_call(p, tp, dis, b_cur.reshape(1, H), W_next)

    p = _agg_call()(srcr, dstr, tp, z64)
    return _fin_call(p, tp, dis, b3.reshape(1, H), batch2, Wf,
                     bf.reshape(1, C))


# speedup vs baseline: 2.3895x; 1.0267x over previous
import functools

import jax
import jax.numpy as jnp
from jax import lax
from jax.experimental import pallas as pl
from jax.experimental.pallas import tpu as pltpu
from jax.experimental.pallas import tpu_sc as plsc

N = 10000
E = 320000
F_IN = 128
H = 64
C = 10
B = 64

NC = 2
NS = 16
L = 16
NW = NC * NS
EPW = E // NW
K = 80
NCH = EPW // K
RING = 10
LOOKAHEAD = 4

ROWS_BLK = 1000
GRID = N // ROWS_BLK


def _deg_body(dst_hbm, z_hbm, out_hbm, acc, ones, dbufs, semd):
    cid = lax.axis_index("c")
    sid = lax.axis_index("s")
    w = cid * NS + sid
    one16 = jnp.ones((L,), jnp.float32)

    @pl.when(sid == 0)
    def _():
        pltpu.sync_copy(z_hbm, acc)

    @pl.loop(0, K)
    def _(i):
        ones[i] = one16

    pltpu.sync_copy(dst_hbm.at[w], dbufs)
    plsc.subcore_barrier()

    def drain_one():
        pltpu.make_async_copy(z_hbm.at[pl.ds(0, K)], ones, semd).wait()

    @pl.loop(0, NCH)
    def _(c):
        @pl.when(c >= 8)
        def _():
            drain_one()
        pltpu.async_copy(ones, acc.at[dbufs.at[c]], semd, add=True)

    for _ in range(8):
        drain_one()

    plsc.subcore_barrier()

    @pl.when(sid == 0)
    def _():
        pltpu.sync_copy(acc, out_hbm.at[cid])


@functools.cache
def _deg_call():
    mesh = plsc.VectorSubcoreMesh(
        core_axis_name="c", subcore_axis_name="s",
        num_cores=NC, num_subcores=NS,
    )
    return pl.kernel(
        _deg_body,
        out_type=jax.ShapeDtypeStruct((NC, N, L), jnp.float32),
        mesh=mesh,
        compiler_params=pltpu.CompilerParams(use_tc_tiling_on_sc=False),
        scratch_types=[
            pltpu.VMEM_SHARED((N, L), jnp.float32),
            pltpu.VMEM((K, L), jnp.float32),
            pltpu.VMEM((NCH, K), jnp.int32),
            pltpu.SemaphoreType.DMA,
        ],
    )


def _agg_body(src_hbm, dst_hbm, tp_hbm, z_hbm, out_hbm,
              acc, sbufs, dbufs, rows, semg, sems):
    cid = lax.axis_index("c")
    sid = lax.axis_index("s")
    w = cid * NS + sid

    @pl.when(sid == 0)
    def _():
        pltpu.sync_copy(z_hbm, acc)

    pltpu.sync_copy(src_hbm.at[w], sbufs)
    pltpu.sync_copy(dst_hbm.at[w], dbufs)
    plsc.subcore_barrier()

    def issue_gather(c, slot):
        pltpu.async_copy(tp_hbm.at[sbufs.at[c]], rows.at[slot], semg)

    def wait_gather():
        pltpu.make_async_copy(tp_hbm.at[pl.ds(0, K)], rows.at[0], semg).wait()

    def drain_scatter():
        pltpu.make_async_copy(tp_hbm.at[pl.ds(0, K)], rows.at[0], sems).wait()

    for c in range(LOOKAHEAD):
        issue_gather(c, c % RING)

    for c in range(NCH):
        nxt = c + LOOKAHEAD
        if nxt < NCH:
            if nxt >= RING:
                drain_scatter()
            issue_gather(nxt, nxt % RING)
        wait_gather()
        pltpu.async_copy(rows.at[c % RING], acc.at[dbufs.at[c]], sems,
                         add=True)

    for _ in range(RING):
        drain_scatter()

    plsc.subcore_barrier()

    @pl.when(sid == 0)
    def _():
        pltpu.sync_copy(acc, out_hbm.at[cid])


@functools.cache
def _agg_call():
    mesh = plsc.VectorSubcoreMesh(
        core_axis_name="c", subcore_axis_name="s",
        num_cores=NC, num_subcores=NS,
    )
    return pl.kernel(
        _agg_body,
        out_type=jax.ShapeDtypeStruct((NC, N, H), jnp.float32),
        mesh=mesh,
        compiler_params=pltpu.CompilerParams(use_tc_tiling_on_sc=False),
        scratch_types=[
            pltpu.VMEM_SHARED((N, H), jnp.float32),
            pltpu.VMEM((NCH, K), jnp.int32),
            pltpu.VMEM((NCH, K), jnp.int32),
            pltpu.VMEM((RING, K, H), jnp.float32),
            pltpu.SemaphoreType.DMA,
            pltpu.SemaphoreType.DMA,
        ],
    )


def _mm_body(x_ref, w_ref, o_ref):
    o_ref[...] = jnp.dot(x_ref[...], w_ref[...],
                         preferred_element_type=jnp.float32)


_mm_call = pl.pallas_call(
    _mm_body,
    grid=(GRID,),
    in_specs=[
        pl.BlockSpec((ROWS_BLK, F_IN), lambda i: (i, 0)),
        pl.BlockSpec((F_IN, H), lambda i: (0, 0)),
    ],
    out_specs=pl.BlockSpec((ROWS_BLK, H), lambda i: (i, 0)),
    out_shape=jax.ShapeDtypeStruct((N, H), jnp.float32),
)


def _scale_body(p_ref, t_ref, dis_ref, tp_ref):
    indeg = p_ref[0, :, 0:1] + p_ref[1, :, 0:1]
    dis = lax.rsqrt(indeg + 1.0)
    dis_ref[...] = dis
    tp_ref[...] = t_ref[...] * dis


_scale_call = pl.pallas_call(
    _scale_body,
    grid=(GRID,),
    in_specs=[
        pl.BlockSpec((NC, ROWS_BLK, L), lambda i: (0, i, 0)),
        pl.BlockSpec((ROWS_BLK, H), lambda i: (i, 0)),
    ],
    out_specs=[
        pl.BlockSpec((ROWS_BLK, 1), lambda i: (i, 0)),
        pl.BlockSpec((ROWS_BLK, H), lambda i: (i, 0)),
    ],
    out_shape=[
        jax.ShapeDtypeStruct((N, 1), jnp.float32),
        jax.ShapeDtypeStruct((N, H), jnp.float32),
    ],
)


def _bnd_body(p_ref, tp_ref, dis_ref, b_ref, w_ref, o_ref):
    dis = dis_ref[...]
    h = jnp.tanh(dis * (p_ref[0] + p_ref[1] + tp_ref[...]) + b_ref[...])
    o_ref[...] = jnp.dot(h, w_ref[...],
                         preferred_element_type=jnp.float32) * dis


_bnd_call = pl.pallas_call(
    _bnd_body,
    grid=(GRID,),
    in_specs=[
        pl.BlockSpec((NC, ROWS_BLK, H), lambda i: (0, i, 0)),
        pl.BlockSpec((ROWS_BLK, H), lambda i: (i, 0)),
        pl.BlockSpec((ROWS_BLK, 1), lambda i: (i, 0)),
        pl.BlockSpec((1, H), lambda i: (0, 0)),
        pl.BlockSpec((H, H), lambda i: (0, 0)),
    ],
    out_specs=pl.BlockSpec((ROWS_BLK, H), lambda i: (i, 0)),
    out_shape=jax.ShapeDtypeStruct((N, H), jnp.float32),
)


def _fin_body(p_ref, tp_ref, dis_ref, b_ref, batch_ref, wf_ref, bf_ref,
              o_ref, acc):
    i = pl.program_id(0)

    @pl.when(i == 0)
    def _():
        acc[...] = jnp.zeros_like(acc)

    h = jnp.tanh(dis_ref[...] * (p_ref[0] + p_ref[1] + tp_ref[...])
                 + b_ref[...])
    seg = lax.broadcasted_iota(jnp.int32, (B, ROWS_BLK), 0)
    onehot = (seg == batch_ref[0]).astype(jnp.float32)
    acc[...] += jnp.dot(onehot, h, preferred_element_type=jnp.float32)

    @pl.when(i == GRID - 1)
    def _():
        o_ref[...] = jnp.tanh(
            jnp.dot(acc[...], wf_ref[...],
                    preferred_element_type=jnp.float32) + bf_ref[...])


_fin_call = pl.pallas_call(
    _fin_body,
    grid=(GRID,),
    in_specs=[
        pl.BlockSpec((NC, ROWS_BLK, H), lambda i: (0, i, 0)),
        pl.BlockSpec((ROWS_BLK, H), lambda i: (i, 0)),
        pl.BlockSpec((ROWS_BLK, 1), lambda i: (i, 0)),
        pl.BlockSpec((1, H), lambda i: (0, 0)),
        pl.BlockSpec((1, 1, ROWS_BLK), lambda i: (i, 0, 0)),
        pl.BlockSpec((H, C), lambda i: (0, 0)),
        pl.BlockSpec((1, C), lambda i: (0, 0)),
    ],
    out_specs=pl.BlockSpec((B, C), lambda i: (0, 0)),
    out_shape=jax.ShapeDtypeStruct((B, C), jnp.float32),
    scratch_shapes=[pltpu.VMEM((B, H), jnp.float32)],
)


def kernel(x, edge_index, batch, W1, b1, W2, b2, W3, b3, Wf, bf):
    srcr = edge_index[0].reshape(NW, NCH, K)
    dstr = edge_index[1].reshape(NW, NCH, K)
    batch2 = batch.reshape(GRID, 1, ROWS_BLK)

    z16 = jnp.zeros((N, L), jnp.float32)
    z64 = jnp.zeros((N, H), jnp.float32)

    degp = _deg_call()(dstr, z16)
    t1 = _mm_call(x, W1)
    dis, tp = _scale_call(degp, t1)

    for W_next, b_cur in ((W2, b1), (W3, b2)):
        p = _agg_call()(srcr, dstr, tp, z64)
        tp = _bnd_call(p, tp, dis, b_cur.reshape(1, H), W_next)

    p = _agg_call()(srcr, dstr, tp, z64)
    return _fin_call(p, tp, dis, b3.reshape(1, H), batch2, Wf,
                     bf.reshape(1, C))
